# baseline (speedup 1.0000x reference)
import functools

import jax
import jax.numpy as jnp
from jax import lax
from jax.experimental import pallas as pl
from jax.experimental.pallas import tpu as pltpu
from jax.experimental.pallas import tpu_sc as plsc

_N = 10000
_E = 320000
_D = 128
_C = 8
_B = 8
_NV = 257

_NC = 2
_NS = 16
_NW = _NC * _NS
_K = 128
_CH = 80
_CHQ = 16
_EPAD = _NW * _CH * _K
_NROWS = 10240
_RPT = _NROWS // _NS
_TRASH = _N

_RH = _NROWS // _NC
_ZSPT = 83968
_TSZ = _NS * _ZSPT
_TRASHF = _RH * _NV
_TCH = _EPAD // _NS // _K
_TSTG = _TCH // _CHQ


def _hist_body(deg_hbm, src_hbm, dst_hbm, t_hbm,
               dv, sidx, didx, fidx, ones, zb1, t_sh, sem):
    c = lax.axis_index("c")
    s = lax.axis_index("s")

    zero16 = jnp.zeros((16,), jnp.float32)
    one16 = jnp.ones((16,), jnp.float32)
    for q in range(2048 // 16):
        zb1[pl.ds(q * 16, 16)] = zero16
    for q in range(_K // 16):
        ones[pl.ds(q * 16, 16)] = one16

    z0 = s * _ZSPT

    @pl.loop(0, _ZSPT // 2048)
    def _zero(i):
        pltpu.sync_copy(zb1, t_sh.at[pl.ds(z0 + i * 2048, 2048)])

    plsc.subcore_barrier()

    base_row = c * _RH

    @pl.loop(0, _TSTG)
    def _stage(q):
        pltpu.sync_copy(src_hbm.at[s, pl.ds(q * _CHQ, _CHQ)], sidx)
        pltpu.sync_copy(dst_hbm.at[s, pl.ds(q * _CHQ, _CHQ)], didx)
        pltpu.async_copy(deg_hbm.at[sidx.at[0]], dv.at[0], sem)
        for j in range(_CHQ):
            b = j % 2
            pltpu.make_async_copy(deg_hbm.at[sidx.at[0]], dv.at[b],
                                  sem).wait()
            if j + 1 < _CHQ:
                pltpu.async_copy(deg_hbm.at[sidx.at[j + 1]], dv.at[1 - b],
                                 sem)
            for g in range(8):
                d16 = didx[j, pl.ds(g * 16, 16)]
                dval = dv[b, pl.ds(g * 16, 16)]
                loc = d16 - base_row
                inr = (loc >= 0) & (loc < _RH)
                flat = jnp.where(inr, loc * _NV + dval, _TRASHF)
                fidx[b, pl.ds(g * 16, 16)] = flat
            pltpu.sync_copy(ones, t_sh.at[fidx.at[b]], add=True)

    plsc.subcore_barrier()

    o0 = s * _ZSPT
    pltpu.sync_copy(t_sh.at[pl.ds(o0, _ZSPT)],
                    t_hbm.at[pl.ds(c * _TSZ + o0, _ZSPT)])


@functools.cache
def _hist_call():
    return pl.kernel(
        _hist_body,
        out_type=jax.ShapeDtypeStruct((_NC * _TSZ,), jnp.float32),
        mesh=plsc.VectorSubcoreMesh(core_axis_name="c", subcore_axis_name="s"),
        scratch_types=[
            pltpu.VMEM((2, _K), jnp.int32),
            pltpu.VMEM((_CHQ, _K), jnp.int32),
            pltpu.VMEM((_CHQ, _K), jnp.int32),
            pltpu.VMEM((2, _K), jnp.int32),
            pltpu.VMEM((_K,), jnp.float32),
            pltpu.VMEM((2048,), jnp.float32),
            pltpu.VMEM_SHARED((_TSZ,), jnp.float32),
            pltpu.SemaphoreType.DMA,
        ],
    )


def _segsum_body(x_hbm, pk_hbm, sum_hbm,
                 ebuf, rows, zbuf, agg_sh, sem):
    c = lax.axis_index("c")
    s = lax.axis_index("s")
    wid = s * _NC + c

    zero16 = jnp.zeros((16,), jnp.float32)
    for r in range(16):
        for q in range(8):
            zbuf[r, pl.ds(q * 16, 16)] = zero16

    r0 = s * _RPT

    @pl.loop(0, _RPT // 16)
    def _zero(i):
        pltpu.sync_copy(zbuf, agg_sh.at[pl.ds(r0 + i * 16, 16)])

    plsc.subcore_barrier()

    base = wid * _CH
    pltpu.sync_copy(pk_hbm.at[base], ebuf.at[0])
    pltpu.async_copy(x_hbm.at[ebuf.at[0, 0]], rows.at[0], sem)

    @pl.loop(0, _CH, step=2)
    def _edges(j):
        for b in range(2):
            jj = j + b

            @pl.when(jj + 1 < _CH)
            def _():
                pltpu.sync_copy(pk_hbm.at[base + jj + 1], ebuf.at[1 - b])

            pltpu.make_async_copy(x_hbm.at[ebuf.at[b, 0]], rows.at[b],
                                  sem).wait()

            @pl.when(jj + 1 < _CH)
            def _():
                pltpu.async_copy(x_hbm.at[ebuf.at[1 - b, 0]], rows.at[1 - b],
                                 sem)

            pltpu.sync_copy(rows.at[b], agg_sh.at[ebuf.at[b, 1]], add=True)

    plsc.subcore_barrier()
    pltpu.sync_copy(agg_sh.at[pl.ds(r0, _RPT)], sum_hbm.at[c, pl.ds(r0, _RPT)])


@functools.cache
def _segsum_call():
    return pl.kernel(
        _segsum_body,
        out_type=jax.ShapeDtypeStruct((_NC, _NROWS, _D), jnp.float32),
        mesh=plsc.VectorSubcoreMesh(core_axis_name="c", subcore_axis_name="s"),
        scratch_types=[
            pltpu.VMEM((2, 2, _K), jnp.int32),
            pltpu.VMEM((2, _K, _D), jnp.float32),
            pltpu.VMEM((16, _D), jnp.float32),
            pltpu.VMEM_SHARED((_NROWS, _D), jnp.float32),
            pltpu.SemaphoreType.DMA,
        ],
    )


def _mmT(a, b):
    return lax.dot_general(a, b, (((1,), (1,)), ((), ())),
                           preferred_element_type=jnp.float32)


def _mm(a, b):
    return lax.dot_general(a, b, (((1,), (0,)), ((), ())),
                           preferred_element_type=jnp.float32)


def _layer0_tc(t0_ref, t1_ref, deg_ref, emb_ref, wl_ref, bl_ref, wr_ref,
               g_ref, be_ref, x_out, r_out):
    emb = emb_ref[...]
    embWl = _mmT(emb, wl_ref[...])
    embWr = _mmT(emb, wr_ref[...])
    rtop = 1.0 / jnp.maximum(
        jnp.sum(t0_ref[...], axis=1, keepdims=True), 1.0)
    rbot = 1.0 / jnp.maximum(
        jnp.sum(t1_ref[...], axis=1, keepdims=True), 1.0)
    topm = _mm(t0_ref[...], embWl) * rtop
    botm = _mm(t1_ref[...], embWl) * rbot
    aggm = jnp.concatenate([topm, botm], axis=0)[:_N, :]
    iota = lax.broadcasted_iota(jnp.int32, (_N, _NV), 1)
    oh = jnp.where(iota == deg_ref[...], 1.0, 0.0)
    xr = _mm(oh, embWr)
    h = aggm + xr + bl_ref[...]
    mean = jnp.mean(h, axis=0, keepdims=True)
    d = h - mean
    var = jnp.mean(d * d, axis=0, keepdims=True)
    y = d * lax.rsqrt(var + 1e-5) * g_ref[...] + be_ref[...]
    x_out[...] = jnp.maximum(y, 0.0)
    r_out[...] = jnp.concatenate([rtop, rbot], axis=0)[:_N, :]


def _layer1_tc(x_ref, parts_ref, r_ref, wl_ref, bl_ref, wr_ref, g_ref,
               be_ref, out_ref):
    agg = parts_ref[0, :_N, :] + parts_ref[1, :_N, :]
    aggm = agg * r_ref[...]
    h = _mmT(aggm, wl_ref[...]) + _mmT(x_ref[...], wr_ref[...]) + bl_ref[...]
    mean = jnp.mean(h, axis=0, keepdims=True)
    d = h - mean
    var = jnp.mean(d * d, axis=0, keepdims=True)
    y = d * lax.rsqrt(var + 1e-5) * g_ref[...] + be_ref[...]
    out_ref[...] = jnp.maximum(y, 0.0)


def _pool_tc(x_ref, batch_ref, wa_ref, ba_ref, wo_ref, bo_ref, out_ref):
    b = pl.program_id(0)
    x = x_ref[...]
    scores = _mmT(x, wa_ref[...]) + ba_ref[...]
    mask = batch_ref[...] == b
    s_i = jnp.where(mask, scores, -1e9)
    m = jnp.max(s_i, axis=0, keepdims=True)
    e = jnp.where(mask, jnp.exp(s_i - m), 0.0)
    denom = jnp.sum(e, axis=0, keepdims=True)
    w = e * (1.0 / jnp.maximum(denom, 1e-30))
    cvec = lax.dot_general(w, x, (((0,), (0,)), ((), ())),
                           preferred_element_type=jnp.float32)
    out_ref[0] = _mmT(cvec, wo_ref[...]) + bo_ref[...]


def _layer0_call(t0, t1, deg2, emb, wl, bl, wr, g, be):
    return pl.pallas_call(
        _layer0_tc,
        out_shape=(jax.ShapeDtypeStruct((_N, _D), jnp.float32),
                   jax.ShapeDtypeStruct((_N, 1), jnp.float32)),
    )(t0, t1, deg2, emb, wl, bl, wr, g, be)


def _layer1_call(x, parts, rvec, wl, bl, wr, g, be):
    return pl.pallas_call(
        _layer1_tc,
        out_shape=jax.ShapeDtypeStruct((_N, _D), jnp.float32),
    )(x, parts, rvec, wl, bl, wr, g, be)


def _pool_call(x, batch2, wa, ba, wo, bo):
    return pl.pallas_call(
        _pool_tc,
        grid=(_B,),
        in_specs=[
            pl.BlockSpec((_N, _D), lambda b: (0, 0)),
            pl.BlockSpec((_N, 1), lambda b: (0, 0)),
            pl.BlockSpec((_C, _D), lambda b: (0, 0)),
            pl.BlockSpec((1, _C), lambda b: (0, 0)),
            pl.BlockSpec((_D, _D), lambda b: (0, 0)),
            pl.BlockSpec((1, _D), lambda b: (0, 0)),
        ],
        out_specs=pl.BlockSpec((1, _C, _D), lambda b: (b, 0, 0)),
        out_shape=jax.ShapeDtypeStruct((_B, _C, _D), jnp.float32),
    )(x, batch2, wa, ba, wo, bo)


def kernel(deg_idx, edge_index, batch, emb, Wl0, bl0, Wr0, g0, be0,
           Wl1, bl1, Wr1, g1, be1, Wa, ba, Wo, bo):
    src = edge_index[0].astype(jnp.int32)
    dst = edge_index[1].astype(jnp.int32)
    npad = _EPAD - _E
    trash = _TRASH + (jnp.arange(npad, dtype=jnp.int32) % (_NROWS - _N))
    src_f = jnp.concatenate([src, jnp.zeros((npad,), jnp.int32)])
    dst_f = jnp.concatenate([dst, trash])
    src_t = src_f.reshape(_NS, _TCH, _K)
    dst_t = dst_f.reshape(_NS, _TCH, _K)
    pk = jnp.stack([src_f.reshape(-1, _K), dst_f.reshape(-1, _K)],
                   axis=1)

    deg = deg_idx.astype(jnp.int32)
    deg2 = deg.reshape(_N, 1)
    batch2 = batch.astype(jnp.int32).reshape(_N, 1)
    bl0r = bl0.reshape(1, _D)
    g0r = g0.reshape(1, _D)
    be0r = be0.reshape(1, _D)
    bl1r = bl1.reshape(1, _D)
    g1r = g1.reshape(1, _D)
    be1r = be1.reshape(1, _D)
    bar = ba.reshape(1, _C)
    bor = bo.reshape(1, _D)

    t_flat = _hist_call()(deg, src_t, dst_t)
    t0 = t_flat[:_RH * _NV].reshape(_RH, _NV)
    t1 = t_flat[_TSZ:_TSZ + _RH * _NV].reshape(_RH, _NV)

    x1, rvec = _layer0_call(t0, t1, deg2, emb, Wl0, bl0r, Wr0, g0r, be0r)
    parts1 = _segsum_call()(x1, pk)
    x2 = _layer1_call(x1, parts1, rvec, Wl1, bl1r, Wr1, g1r, be1r)
    return _pool_call(x2, batch2, Wa, bar, Wo, bor)


# speedup vs baseline: 2.0373x; 2.0373x over previous
import functools

import jax
import jax.numpy as jnp
from jax import lax
from jax.experimental import pallas as pl
from jax.experimental.pallas import tpu as pltpu
from jax.experimental.pallas import tpu_sc as plsc

_N = 10000
_E = 320000
_D = 128
_C = 8
_B = 8
_NV = 257

_NC = 2
_NS = 16
_NW = _NC * _NS
_K = 128
_CH = 80
_CHQ = 16
_EPAD = _NW * _CH * _K
_NROWS = 10240
_RPT = _NROWS // _NS
_TRASH = _N

_RH = _NROWS // _NC
_ZSPT = 83968
_TSZ = _NS * _ZSPT
_TRASHF = _RH * _NV
_TCH = _EPAD // _NS // _K
_TSTG = _TCH // _CHQ


def _hist_body(deg_hbm, src_hbm, dst_hbm, t_hbm,
               dv, sidx, didx, fidx, ones, zb1, t_sh, sem):
    c = lax.axis_index("c")
    s = lax.axis_index("s")

    zero16 = jnp.zeros((16,), jnp.float32)
    one16 = jnp.ones((16,), jnp.float32)
    for q in range(2048 // 16):
        zb1[pl.ds(q * 16, 16)] = zero16
    for q in range(_K // 16):
        ones[pl.ds(q * 16, 16)] = one16

    z0 = s * _ZSPT

    @pl.loop(0, _ZSPT // 2048)
    def _zero(i):
        pltpu.sync_copy(zb1, t_sh.at[pl.ds(z0 + i * 2048, 2048)])

    plsc.subcore_barrier()

    base_row = c * _RH

    @pl.loop(0, _TSTG)
    def _stage(q):
        pltpu.sync_copy(src_hbm.at[s, pl.ds(q * _CHQ, _CHQ)], sidx)
        pltpu.sync_copy(dst_hbm.at[s, pl.ds(q * _CHQ, _CHQ)], didx)
        pltpu.async_copy(deg_hbm.at[sidx.at[0]], dv.at[0], sem)
        for j in range(_CHQ):
            b = j % 2
            pltpu.make_async_copy(deg_hbm.at[sidx.at[0]], dv.at[b],
                                  sem).wait()
            if j + 1 < _CHQ:
                pltpu.async_copy(deg_hbm.at[sidx.at[j + 1]], dv.at[1 - b],
                                 sem)
            for g in range(8):
                d16 = didx[j, pl.ds(g * 16, 16)]
                dval = dv[b, pl.ds(g * 16, 16)]
                loc = d16 - base_row
                inr = (loc >= 0) & (loc < _RH)
                flat = jnp.where(inr, loc * _NV + dval, _TRASHF)
                fidx[b, pl.ds(g * 16, 16)] = flat
            pltpu.sync_copy(ones, t_sh.at[fidx.at[b]], add=True)

    plsc.subcore_barrier()

    o0 = s * _ZSPT
    pltpu.sync_copy(t_sh.at[pl.ds(o0, _ZSPT)],
                    t_hbm.at[pl.ds(c * _TSZ + o0, _ZSPT)])


@functools.cache
def _hist_call():
    return pl.kernel(
        _hist_body,
        out_type=jax.ShapeDtypeStruct((_NC * _TSZ,), jnp.float32),
        mesh=plsc.VectorSubcoreMesh(core_axis_name="c", subcore_axis_name="s"),
        scratch_types=[
            pltpu.VMEM((2, _K), jnp.int32),
            pltpu.VMEM((_CHQ, _K), jnp.int32),
            pltpu.VMEM((_CHQ, _K), jnp.int32),
            pltpu.VMEM((2, _K), jnp.int32),
            pltpu.VMEM((_K,), jnp.float32),
            pltpu.VMEM((2048,), jnp.float32),
            pltpu.VMEM_SHARED((_TSZ,), jnp.float32),
            pltpu.SemaphoreType.DMA,
        ],
    )


def _segsum_body(x_hbm, pk_hbm, sum_hbm,
                 ebuf, rows, zbuf, agg_sh, sem):
    c = lax.axis_index("c")
    s = lax.axis_index("s")
    wid = s * _NC + c

    zero16 = jnp.zeros((16,), jnp.float32)
    for r in range(16):
        for q in range(8):
            zbuf[r, pl.ds(q * 16, 16)] = zero16

    r0 = s * _RPT

    @pl.loop(0, _RPT // 16)
    def _zero(i):
        pltpu.sync_copy(zbuf, agg_sh.at[pl.ds(r0 + i * 16, 16)])

    plsc.subcore_barrier()

    base = wid * _CH
    pltpu.sync_copy(pk_hbm.at[base], ebuf.at[0])
    pltpu.async_copy(x_hbm.at[ebuf.at[0, 0]], rows.at[0], sem)

    @pl.loop(0, _CH, step=2)
    def _edges(j):
        for b in range(2):
            jj = j + b

            @pl.when(jj + 1 < _CH)
            def _():
                pltpu.sync_copy(pk_hbm.at[base + jj + 1], ebuf.at[1 - b])

            pltpu.make_async_copy(x_hbm.at[ebuf.at[b, 0]], rows.at[b],
                                  sem).wait()

            @pl.when(jj + 1 < _CH)
            def _():
                pltpu.async_copy(x_hbm.at[ebuf.at[1 - b, 0]], rows.at[1 - b],
                                 sem)

            pltpu.sync_copy(rows.at[b], agg_sh.at[ebuf.at[b, 1]], add=True)

    plsc.subcore_barrier()
    pltpu.sync_copy(agg_sh.at[pl.ds(r0, _RPT)], sum_hbm.at[c, pl.ds(r0, _RPT)])


@functools.cache
def _segsum_call():
    return pl.kernel(
        _segsum_body,
        out_type=jax.ShapeDtypeStruct((_NC, _NROWS, _D), jnp.float32),
        mesh=plsc.VectorSubcoreMesh(core_axis_name="c", subcore_axis_name="s"),
        scratch_types=[
            pltpu.VMEM((2, 2, _K), jnp.int32),
            pltpu.VMEM((2, _K, _D), jnp.float32),
            pltpu.VMEM((16, _D), jnp.float32),
            pltpu.VMEM_SHARED((_NROWS, _D), jnp.float32),
            pltpu.SemaphoreType.DMA,
        ],
    )


def _mmT(a, b):
    return lax.dot_general(a, b, (((1,), (1,)), ((), ())),
                           preferred_element_type=jnp.float32)


def _mm(a, b):
    return lax.dot_general(a, b, (((1,), (0,)), ((), ())),
                           preferred_element_type=jnp.float32)


def _layer0_tc(t0_ref, t1_ref, deg_ref, emb_ref, wl_ref, bl_ref, wr_ref,
               g_ref, be_ref, x_out, r_out):
    emb = emb_ref[...]
    embWl = _mmT(emb, wl_ref[...])
    embWr = _mmT(emb, wr_ref[...])
    rtop = 1.0 / jnp.maximum(
        jnp.sum(t0_ref[...], axis=1, keepdims=True), 1.0)
    rbot = 1.0 / jnp.maximum(
        jnp.sum(t1_ref[...], axis=1, keepdims=True), 1.0)
    topm = _mm(t0_ref[...], embWl) * rtop
    botm = _mm(t1_ref[...], embWl) * rbot
    aggm = jnp.concatenate([topm, botm], axis=0)[:_N, :]
    iota = lax.broadcasted_iota(jnp.int32, (_N, _NV), 1)
    oh = jnp.where(iota == deg_ref[...], 1.0, 0.0)
    xr = _mm(oh, embWr)
    h = aggm + xr + bl_ref[...]
    mean = jnp.mean(h, axis=0, keepdims=True)
    d = h - mean
    var = jnp.mean(d * d, axis=0, keepdims=True)
    y = d * lax.rsqrt(var + 1e-5) * g_ref[...] + be_ref[...]
    x_out[...] = jnp.maximum(y, 0.0)
    r_out[...] = jnp.concatenate([rtop, rbot], axis=0)[:_N, :]


def _layer1_tc(x_ref, parts_ref, r_ref, wl_ref, bl_ref, wr_ref, g_ref,
               be_ref, out_ref):
    agg = parts_ref[0, :_N, :] + parts_ref[1, :_N, :]
    aggm = agg * r_ref[...]
    h = _mmT(aggm, wl_ref[...]) + _mmT(x_ref[...], wr_ref[...]) + bl_ref[...]
    mean = jnp.mean(h, axis=0, keepdims=True)
    d = h - mean
    var = jnp.mean(d * d, axis=0, keepdims=True)
    y = d * lax.rsqrt(var + 1e-5) * g_ref[...] + be_ref[...]
    out_ref[...] = jnp.maximum(y, 0.0)


def _pool_tc(x_ref, batch_ref, wa_ref, ba_ref, wo_ref, bo_ref, out_ref):
    b = pl.program_id(0)
    x = x_ref[...]
    scores = _mmT(x, wa_ref[...]) + ba_ref[...]
    mask = batch_ref[...] == b
    s_i = jnp.where(mask, scores, -1e9)
    m = jnp.max(s_i, axis=0, keepdims=True)
    e = jnp.where(mask, jnp.exp(s_i - m), 0.0)
    denom = jnp.sum(e, axis=0, keepdims=True)
    w = e * (1.0 / jnp.maximum(denom, 1e-30))
    cvec = lax.dot_general(w, x, (((0,), (0,)), ((), ())),
                           preferred_element_type=jnp.float32)
    out_ref[0] = _mmT(cvec, wo_ref[...]) + bo_ref[...]


def _layer0_call(t0, t1, deg2, emb, wl, bl, wr, g, be):
    return pl.pallas_call(
        _layer0_tc,
        out_shape=(jax.ShapeDtypeStruct((_N, _D), jnp.float32),
                   jax.ShapeDtypeStruct((_N, 1), jnp.float32)),
    )(t0, t1, deg2, emb, wl, bl, wr, g, be)


def _layer1_call(x, parts, rvec, wl, bl, wr, g, be):
    return pl.pallas_call(
        _layer1_tc,
        out_shape=jax.ShapeDtypeStruct((_N, _D), jnp.float32),
    )(x, parts, rvec, wl, bl, wr, g, be)


def _pool_call(x, batch2, wa, ba, wo, bo):
    return pl.pallas_call(
        _pool_tc,
        grid=(_B,),
        in_specs=[
            pl.BlockSpec((_N, _D), lambda b: (0, 0)),
            pl.BlockSpec((_N, 1), lambda b: (0, 0)),
            pl.BlockSpec((_C, _D), lambda b: (0, 0)),
            pl.BlockSpec((1, _C), lambda b: (0, 0)),
            pl.BlockSpec((_D, _D), lambda b: (0, 0)),
            pl.BlockSpec((1, _D), lambda b: (0, 0)),
        ],
        out_specs=pl.BlockSpec((1, _C, _D), lambda b: (b, 0, 0)),
        out_shape=jax.ShapeDtypeStruct((_B, _C, _D), jnp.float32),
    )(x, batch2, wa, ba, wo, bo)


def kernel(deg_idx, edge_index, batch, emb, Wl0, bl0, Wr0, g0, be0,
           Wl1, bl1, Wr1, g1, be1, Wa, ba, Wo, bo):
    src = edge_index[0].astype(jnp.int32)
    dst = edge_index[1].astype(jnp.int32)
    npad = _EPAD - _E
    ppw = npad // _NW
    rpw = _E // _NW
    pad_src = (jnp.arange(npad, dtype=jnp.int32) * 37) % _N
    pad_dst = _TRASH + (jnp.arange(npad, dtype=jnp.int32) % (_NROWS - _N))
    src_f = jnp.concatenate([src.reshape(_NW, rpw),
                             pad_src.reshape(_NW, ppw)], axis=1).reshape(-1)
    dst_f = jnp.concatenate([dst.reshape(_NW, rpw),
                             pad_dst.reshape(_NW, ppw)], axis=1).reshape(-1)
    src_t = src_f.reshape(_NS, _TCH, _K)
    dst_t = dst_f.reshape(_NS, _TCH, _K)
    pk = jnp.stack([src_f.reshape(-1, _K), dst_f.reshape(-1, _K)],
                   axis=1)

    deg = deg_idx.astype(jnp.int32)
    deg2 = deg.reshape(_N, 1)
    batch2 = batch.astype(jnp.int32).reshape(_N, 1)
    bl0r = bl0.reshape(1, _D)
    g0r = g0.reshape(1, _D)
    be0r = be0.reshape(1, _D)
    bl1r = bl1.reshape(1, _D)
    g1r = g1.reshape(1, _D)
    be1r = be1.reshape(1, _D)
    bar = ba.reshape(1, _C)
    bor = bo.reshape(1, _D)

    t_flat = _hist_call()(deg, src_t, dst_t)
    t0 = t_flat[:_RH * _NV].reshape(_RH, _NV)
    t1 = t_flat[_TSZ:_TSZ + _RH * _NV].reshape(_RH, _NV)

    x1, rvec = _layer0_call(t0, t1, deg2, emb, Wl0, bl0r, Wr0, g0r, be0r)
    parts1 = _segsum_call()(x1, pk)
    x2 = _layer1_call(x1, parts1, rvec, Wl1, bl1r, Wr1, g1r, be1r)
    return _pool_call(x2, batch2, Wa, bar, Wo, bor)


# speedup vs baseline: 2.0423x; 1.0024x over previous
import functools

import jax
import jax.numpy as jnp
from jax import lax
from jax.experimental import pallas as pl
from jax.experimental.pallas import tpu as pltpu
from jax.experimental.pallas import tpu_sc as plsc

_N = 10000
_E = 320000
_D = 128
_C = 8
_B = 8
_NV = 257

_NC = 2
_NS = 16
_NW = _NC * _NS
_K = 128
_CH = 80
_CHQ = 16
_EPAD = _NW * _CH * _K
_NROWS = 10240
_RPT = _NROWS // _NS
_TRASH = _N

_RH = _NROWS // _NC
_ZSPT = 83968
_TSZ = _NS * _ZSPT
_TRASHF = _RH * _NV
_TCH = _EPAD // _NS // _K
_TSTG = _TCH // _CHQ


def _hist_body(deg_hbm, src_hbm, dst_hbm, t_hbm,
               dv, sidx, didx, fidx, ones, zb1, t_sh, sem, ssem):
    c = lax.axis_index("c")
    s = lax.axis_index("s")

    zero16 = jnp.zeros((16,), jnp.float32)
    one16 = jnp.ones((16,), jnp.float32)
    for q in range(2048 // 16):
        zb1[pl.ds(q * 16, 16)] = zero16
    for q in range(_K // 16):
        ones[pl.ds(q * 16, 16)] = one16

    z0 = s * _ZSPT

    @pl.loop(0, _ZSPT // 2048)
    def _zero(i):
        pltpu.sync_copy(zb1, t_sh.at[pl.ds(z0 + i * 2048, 2048)])

    plsc.subcore_barrier()

    base_row = c * _RH

    @pl.loop(0, _TSTG)
    def _stage(q):
        pltpu.sync_copy(src_hbm.at[s, pl.ds(q * _CHQ, _CHQ)], sidx)
        pltpu.sync_copy(dst_hbm.at[s, pl.ds(q * _CHQ, _CHQ)], didx)
        pltpu.async_copy(deg_hbm.at[sidx.at[0]], dv.at[0], sem)
        for j in range(_CHQ):
            b = j % 2
            f4 = j % 4
            pltpu.make_async_copy(deg_hbm.at[sidx.at[0]], dv.at[b],
                                  sem).wait()
            if j + 1 < _CHQ:
                pltpu.async_copy(deg_hbm.at[sidx.at[j + 1]], dv.at[1 - b],
                                 sem)
            if j >= 4:
                pltpu.make_async_copy(ones, t_sh.at[fidx.at[f4]],
                                      ssem).wait()
            for g in range(8):
                d16 = didx[j, pl.ds(g * 16, 16)]
                dval = dv[b, pl.ds(g * 16, 16)]
                loc = d16 - base_row
                inr = (loc >= 0) & (loc < _RH)
                flat = jnp.where(inr, loc * _NV + dval, _TRASHF)
                fidx[f4, pl.ds(g * 16, 16)] = flat
            pltpu.async_copy(ones, t_sh.at[fidx.at[f4]], ssem, add=True)
        for f4 in range(4):
            pltpu.make_async_copy(ones, t_sh.at[fidx.at[f4]], ssem).wait()

    plsc.subcore_barrier()

    o0 = s * _ZSPT
    pltpu.sync_copy(t_sh.at[pl.ds(o0, _ZSPT)],
                    t_hbm.at[pl.ds(c * _TSZ + o0, _ZSPT)])


@functools.cache
def _hist_call():
    return pl.kernel(
        _hist_body,
        out_type=jax.ShapeDtypeStruct((_NC * _TSZ,), jnp.float32),
        mesh=plsc.VectorSubcoreMesh(core_axis_name="c", subcore_axis_name="s"),
        scratch_types=[
            pltpu.VMEM((2, _K), jnp.int32),
            pltpu.VMEM((_CHQ, _K), jnp.int32),
            pltpu.VMEM((_CHQ, _K), jnp.int32),
            pltpu.VMEM((4, _K), jnp.int32),
            pltpu.VMEM((_K,), jnp.float32),
            pltpu.VMEM((2048,), jnp.float32),
            pltpu.VMEM_SHARED((_TSZ,), jnp.float32),
            pltpu.SemaphoreType.DMA,
            pltpu.SemaphoreType.DMA,
        ],
    )


def _segsum_body(x_hbm, pk_hbm, sum_hbm,
                 ebuf, rows, zbuf, agg_sh, sem):
    c = lax.axis_index("c")
    s = lax.axis_index("s")
    wid = s * _NC + c

    zero16 = jnp.zeros((16,), jnp.float32)
    for r in range(16):
        for q in range(8):
            zbuf[r, pl.ds(q * 16, 16)] = zero16

    r0 = s * _RPT

    @pl.loop(0, _RPT // 16)
    def _zero(i):
        pltpu.sync_copy(zbuf, agg_sh.at[pl.ds(r0 + i * 16, 16)])

    plsc.subcore_barrier()

    base = wid * _CH
    pltpu.sync_copy(pk_hbm.at[base], ebuf.at[0])
    pltpu.async_copy(x_hbm.at[ebuf.at[0, 0]], rows.at[0], sem)

    @pl.loop(0, _CH, step=2)
    def _edges(j):
        for b in range(2):
            jj = j + b

            @pl.when(jj + 1 < _CH)
            def _():
                pltpu.sync_copy(pk_hbm.at[base + jj + 1], ebuf.at[1 - b])

            pltpu.make_async_copy(x_hbm.at[ebuf.at[b, 0]], rows.at[b],
                                  sem).wait()

            @pl.when(jj + 1 < _CH)
            def _():
                pltpu.async_copy(x_hbm.at[ebuf.at[1 - b, 0]], rows.at[1 - b],
                                 sem)

            pltpu.sync_copy(rows.at[b], agg_sh.at[ebuf.at[b, 1]], add=True)

    plsc.subcore_barrier()
    pltpu.sync_copy(agg_sh.at[pl.ds(r0, _RPT)], sum_hbm.at[c, pl.ds(r0, _RPT)])


@functools.cache
def _segsum_call():
    return pl.kernel(
        _segsum_body,
        out_type=jax.ShapeDtypeStruct((_NC, _NROWS, _D), jnp.float32),
        mesh=plsc.VectorSubcoreMesh(core_axis_name="c", subcore_axis_name="s"),
        scratch_types=[
            pltpu.VMEM((2, 2, _K), jnp.int32),
            pltpu.VMEM((2, _K, _D), jnp.float32),
            pltpu.VMEM((16, _D), jnp.float32),
            pltpu.VMEM_SHARED((_NROWS, _D), jnp.float32),
            pltpu.SemaphoreType.DMA,
        ],
    )


def _mmT(a, b):
    return lax.dot_general(a, b, (((1,), (1,)), ((), ())),
                           preferred_element_type=jnp.float32)


def _mm(a, b):
    return lax.dot_general(a, b, (((1,), (0,)), ((), ())),
                           preferred_element_type=jnp.float32)


def _layer0_tc(t0_ref, t1_ref, deg_ref, emb_ref, wl_ref, bl_ref, wr_ref,
               g_ref, be_ref, x_out, r_out):
    emb = emb_ref[...]
    embWl = _mmT(emb, wl_ref[...])
    embWr = _mmT(emb, wr_ref[...])
    rtop = 1.0 / jnp.maximum(
        jnp.sum(t0_ref[...], axis=1, keepdims=True), 1.0)
    rbot = 1.0 / jnp.maximum(
        jnp.sum(t1_ref[...], axis=1, keepdims=True), 1.0)
    topm = _mm(t0_ref[...], embWl) * rtop
    botm = _mm(t1_ref[...], embWl) * rbot
    aggm = jnp.concatenate([topm, botm], axis=0)[:_N, :]
    iota = lax.broadcasted_iota(jnp.int32, (_N, _NV), 1)
    oh = jnp.where(iota == deg_ref[...], 1.0, 0.0)
    xr = _mm(oh, embWr)
    h = aggm + xr + bl_ref[...]
    mean = jnp.mean(h, axis=0, keepdims=True)
    d = h - mean
    var = jnp.mean(d * d, axis=0, keepdims=True)
    y = d * lax.rsqrt(var + 1e-5) * g_ref[...] + be_ref[...]
    x_out[...] = jnp.maximum(y, 0.0)
    r_out[...] = jnp.concatenate([rtop, rbot], axis=0)[:_N, :]


def _layer1_tc(x_ref, parts_ref, r_ref, wl_ref, bl_ref, wr_ref, g_ref,
               be_ref, out_ref):
    agg = parts_ref[0, :_N, :] + parts_ref[1, :_N, :]
    aggm = agg * r_ref[...]
    h = _mmT(aggm, wl_ref[...]) + _mmT(x_ref[...], wr_ref[...]) + bl_ref[...]
    mean = jnp.mean(h, axis=0, keepdims=True)
    d = h - mean
    var = jnp.mean(d * d, axis=0, keepdims=True)
    y = d * lax.rsqrt(var + 1e-5) * g_ref[...] + be_ref[...]
    out_ref[...] = jnp.maximum(y, 0.0)


def _pool_tc(x_ref, batch_ref, wa_ref, ba_ref, wo_ref, bo_ref, out_ref):
    b = pl.program_id(0)
    x = x_ref[...]
    scores = _mmT(x, wa_ref[...]) + ba_ref[...]
    mask = batch_ref[...] == b
    s_i = jnp.where(mask, scores, -1e9)
    m = jnp.max(s_i, axis=0, keepdims=True)
    e = jnp.where(mask, jnp.exp(s_i - m), 0.0)
    denom = jnp.sum(e, axis=0, keepdims=True)
    w = e * (1.0 / jnp.maximum(denom, 1e-30))
    cvec = lax.dot_general(w, x, (((0,), (0,)), ((), ())),
                           preferred_element_type=jnp.float32)
    out_ref[0] = _mmT(cvec, wo_ref[...]) + bo_ref[...]


def _layer0_call(t0, t1, deg2, emb, wl, bl, wr, g, be):
    return pl.pallas_call(
        _layer0_tc,
        out_shape=(jax.ShapeDtypeStruct((_N, _D), jnp.float32),
                   jax.ShapeDtypeStruct((_N, 1), jnp.float32)),
    )(t0, t1, deg2, emb, wl, bl, wr, g, be)


def _layer1_call(x, parts, rvec, wl, bl, wr, g, be):
    return pl.pallas_call(
        _layer1_tc,
        out_shape=jax.ShapeDtypeStruct((_N, _D), jnp.float32),
    )(x, parts, rvec, wl, bl, wr, g, be)


def _pool_call(x, batch2, wa, ba, wo, bo):
    return pl.pallas_call(
        _pool_tc,
        grid=(_B,),
        in_specs=[
            pl.BlockSpec((_N, _D), lambda b: (0, 0)),
            pl.BlockSpec((_N, 1), lambda b: (0, 0)),
            pl.BlockSpec((_C, _D), lambda b: (0, 0)),
            pl.BlockSpec((1, _C), lambda b: (0, 0)),
            pl.BlockSpec((_D, _D), lambda b: (0, 0)),
            pl.BlockSpec((1, _D), lambda b: (0, 0)),
        ],
        out_specs=pl.BlockSpec((1, _C, _D), lambda b: (b, 0, 0)),
        out_shape=jax.ShapeDtypeStruct((_B, _C, _D), jnp.float32),
    )(x, batch2, wa, ba, wo, bo)


def kernel(deg_idx, edge_index, batch, emb, Wl0, bl0, Wr0, g0, be0,
           Wl1, bl1, Wr1, g1, be1, Wa, ba, Wo, bo):
    src = edge_index[0].astype(jnp.int32)
    dst = edge_index[1].astype(jnp.int32)
    npad = _EPAD - _E
    ppw = npad // _NW
    rpw = _E // _NW
    pad_src = (jnp.arange(npad, dtype=jnp.int32) * 37) % _N
    pad_dst = _TRASH + (jnp.arange(npad, dtype=jnp.int32) % (_NROWS - _N))
    src_f = jnp.concatenate([src.reshape(_NW, rpw),
                             pad_src.reshape(_NW, ppw)], axis=1).reshape(-1)
    dst_f = jnp.concatenate([dst.reshape(_NW, rpw),
                             pad_dst.reshape(_NW, ppw)], axis=1).reshape(-1)
    src_t = src_f.reshape(_NS, _TCH, _K)
    dst_t = dst_f.reshape(_NS, _TCH, _K)
    pk = jnp.stack([src_f.reshape(-1, _K), dst_f.reshape(-1, _K)],
                   axis=1)

    deg = deg_idx.astype(jnp.int32)
    deg2 = deg.reshape(_N, 1)
    batch2 = batch.astype(jnp.int32).reshape(_N, 1)
    bl0r = bl0.reshape(1, _D)
    g0r = g0.reshape(1, _D)
    be0r = be0.reshape(1, _D)
    bl1r = bl1.reshape(1, _D)
    g1r = g1.reshape(1, _D)
    be1r = be1.reshape(1, _D)
    bar = ba.reshape(1, _C)
    bor = bo.reshape(1, _D)

    t_flat = _hist_call()(deg, src_t, dst_t)
    t0 = t_flat[:_RH * _NV].reshape(_RH, _NV)
    t1 = t_flat[_TSZ:_TSZ + _RH * _NV].reshape(_RH, _NV)

    x1, rvec = _layer0_call(t0, t1, deg2, emb, Wl0, bl0r, Wr0, g0r, be0r)
    parts1 = _segsum_call()(x1, pk)
    x2 = _layer1_call(x1, parts1, rvec, Wl1, bl1r, Wr1, g1r, be1r)
    return _pool_call(x2, batch2, Wa, bar, Wo, bor)


# speedup vs baseline: 2.0441x; 1.0009x over previous
import functools

import jax
import jax.numpy as jnp
from jax import lax
from jax.experimental import pallas as pl
from jax.experimental.pallas import tpu as pltpu
from jax.experimental.pallas import tpu_sc as plsc

_N = 10000
_E = 320000
_D = 128
_C = 8
_B = 8
_NV = 257

_NC = 2
_NS = 16
_NW = _NC * _NS
_K = 128
_CH = 80
_CHQ = 16
_EPAD = _NW * _CH * _K
_NROWS = 10240
_RPT = _NROWS // _NS
_TRASH = _N

_RH = _NROWS // _NC
_ZSPT = 83968
_TSZ = _NS * _ZSPT
_TRASHF = _RH * _NV
_TCH = _EPAD // _NS // _K
_TSTG = _TCH // _CHQ


def _hist_body(deg_hbm, src_hbm, dst_hbm, t_hbm,
               dv, sidx, didx, fidx, ones, zb1, t_sh, sem, ssem):
    c = lax.axis_index("c")
    s = lax.axis_index("s")

    zero16 = jnp.zeros((16,), jnp.float32)
    one16 = jnp.ones((16,), jnp.float32)
    for q in range(2048 // 16):
        zb1[pl.ds(q * 16, 16)] = zero16
    for q in range(_K // 16):
        ones[pl.ds(q * 16, 16)] = one16

    z0 = s * _ZSPT

    @pl.loop(0, _ZSPT // 2048)
    def _zero(i):
        pltpu.sync_copy(zb1, t_sh.at[pl.ds(z0 + i * 2048, 2048)])

    plsc.subcore_barrier()

    base_row = c * _RH

    @pl.loop(0, _TSTG)
    def _stage(q):
        pltpu.sync_copy(src_hbm.at[s, pl.ds(q * _CHQ, _CHQ)], sidx)
        pltpu.sync_copy(dst_hbm.at[s, pl.ds(q * _CHQ, _CHQ)], didx)
        for p in range(4):
            pltpu.async_copy(deg_hbm.at[sidx.at[p]], dv.at[p], sem)
        for j in range(_CHQ):
            f4 = j % 4
            pltpu.make_async_copy(deg_hbm.at[sidx.at[0]], dv.at[f4],
                                  sem).wait()
            if j >= 4:
                pltpu.make_async_copy(ones, t_sh.at[fidx.at[f4]],
                                      ssem).wait()
            for g in range(8):
                d16 = didx[j, pl.ds(g * 16, 16)]
                dval = dv[f4, pl.ds(g * 16, 16)]
                loc = d16 - base_row
                inr = (loc >= 0) & (loc < _RH)
                flat = jnp.where(inr, loc * _NV + dval, _TRASHF)
                fidx[f4, pl.ds(g * 16, 16)] = flat
            pltpu.async_copy(ones, t_sh.at[fidx.at[f4]], ssem, add=True)
            if j + 4 < _CHQ:
                pltpu.async_copy(deg_hbm.at[sidx.at[j + 4]], dv.at[f4], sem)
        for f4 in range(4):
            pltpu.make_async_copy(ones, t_sh.at[fidx.at[f4]], ssem).wait()

    plsc.subcore_barrier()

    o0 = s * _ZSPT
    pltpu.sync_copy(t_sh.at[pl.ds(o0, _ZSPT)],
                    t_hbm.at[pl.ds(c * _TSZ + o0, _ZSPT)])


@functools.cache
def _hist_call():
    return pl.kernel(
        _hist_body,
        out_type=jax.ShapeDtypeStruct((_NC * _TSZ,), jnp.float32),
        mesh=plsc.VectorSubcoreMesh(core_axis_name="c", subcore_axis_name="s"),
        scratch_types=[
            pltpu.VMEM((4, _K), jnp.int32),
            pltpu.VMEM((_CHQ, _K), jnp.int32),
            pltpu.VMEM((_CHQ, _K), jnp.int32),
            pltpu.VMEM((4, _K), jnp.int32),
            pltpu.VMEM((_K,), jnp.float32),
            pltpu.VMEM((2048,), jnp.float32),
            pltpu.VMEM_SHARED((_TSZ,), jnp.float32),
            pltpu.SemaphoreType.DMA,
            pltpu.SemaphoreType.DMA,
        ],
    )


def _segsum_body(x_hbm, pk_hbm, sum_hbm,
                 ebuf, rows, zbuf, agg_sh, sem):
    c = lax.axis_index("c")
    s = lax.axis_index("s")
    wid = s * _NC + c

    zero16 = jnp.zeros((16,), jnp.float32)
    for r in range(16):
        for q in range(8):
            zbuf[r, pl.ds(q * 16, 16)] = zero16

    r0 = s * _RPT

    @pl.loop(0, _RPT // 16)
    def _zero(i):
        pltpu.sync_copy(zbuf, agg_sh.at[pl.ds(r0 + i * 16, 16)])

    plsc.subcore_barrier()

    base = wid * _CH
    pltpu.sync_copy(pk_hbm.at[base], ebuf.at[0])
    pltpu.async_copy(x_hbm.at[ebuf.at[0, 0]], rows.at[0], sem)

    @pl.loop(0, _CH, step=2)
    def _edges(j):
        for b in range(2):
            jj = j + b

            @pl.when(jj + 1 < _CH)
            def _():
                pltpu.sync_copy(pk_hbm.at[base + jj + 1], ebuf.at[1 - b])

            pltpu.make_async_copy(x_hbm.at[ebuf.at[b, 0]], rows.at[b],
                                  sem).wait()

            @pl.when(jj + 1 < _CH)
            def _():
                pltpu.async_copy(x_hbm.at[ebuf.at[1 - b, 0]], rows.at[1 - b],
                                 sem)

            pltpu.sync_copy(rows.at[b], agg_sh.at[ebuf.at[b, 1]], add=True)

    plsc.subcore_barrier()
    pltpu.sync_copy(agg_sh.at[pl.ds(r0, _RPT)], sum_hbm.at[c, pl.ds(r0, _RPT)])


@functools.cache
def _segsum_call():
    return pl.kernel(
        _segsum_body,
        out_type=jax.ShapeDtypeStruct((_NC, _NROWS, _D), jnp.float32),
        mesh=plsc.VectorSubcoreMesh(core_axis_name="c", subcore_axis_name="s"),
        scratch_types=[
            pltpu.VMEM((2, 2, _K), jnp.int32),
            pltpu.VMEM((2, _K, _D), jnp.float32),
            pltpu.VMEM((16, _D), jnp.float32),
            pltpu.VMEM_SHARED((_NROWS, _D), jnp.float32),
            pltpu.SemaphoreType.DMA,
        ],
    )


def _mmT(a, b):
    return lax.dot_general(a, b, (((1,), (1,)), ((), ())),
                           preferred_element_type=jnp.float32)


def _mm(a, b):
    return lax.dot_general(a, b, (((1,), (0,)), ((), ())),
                           preferred_element_type=jnp.float32)


def _layer0_tc(t0_ref, t1_ref, deg_ref, emb_ref, wl_ref, bl_ref, wr_ref,
               g_ref, be_ref, x_out, r_out):
    emb = emb_ref[...]
    embWl = _mmT(emb, wl_ref[...])
    embWr = _mmT(emb, wr_ref[...])
    rtop = 1.0 / jnp.maximum(
        jnp.sum(t0_ref[...], axis=1, keepdims=True), 1.0)
    rbot = 1.0 / jnp.maximum(
        jnp.sum(t1_ref[...], axis=1, keepdims=True), 1.0)
    topm = _mm(t0_ref[...], embWl) * rtop
    botm = _mm(t1_ref[...], embWl) * rbot
    aggm = jnp.concatenate([topm, botm], axis=0)[:_N, :]
    iota = lax.broadcasted_iota(jnp.int32, (_N, _NV), 1)
    oh = jnp.where(iota == deg_ref[...], 1.0, 0.0)
    xr = _mm(oh, embWr)
    h = aggm + xr + bl_ref[...]
    mean = jnp.mean(h, axis=0, keepdims=True)
    d = h - mean
    var = jnp.mean(d * d, axis=0, keepdims=True)
    y = d * lax.rsqrt(var + 1e-5) * g_ref[...] + be_ref[...]
    x_out[...] = jnp.maximum(y, 0.0)
    r_out[...] = jnp.concatenate([rtop, rbot], axis=0)[:_N, :]


def _layer1_tc(x_ref, parts_ref, r_ref, wl_ref, bl_ref, wr_ref, g_ref,
               be_ref, out_ref):
    agg = parts_ref[0, :_N, :] + parts_ref[1, :_N, :]
    aggm = agg * r_ref[...]
    h = _mmT(aggm, wl_ref[...]) + _mmT(x_ref[...], wr_ref[...]) + bl_ref[...]
    mean = jnp.mean(h, axis=0, keepdims=True)
    d = h - mean
    var = jnp.mean(d * d, axis=0, keepdims=True)
    y = d * lax.rsqrt(var + 1e-5) * g_ref[...] + be_ref[...]
    out_ref[...] = jnp.maximum(y, 0.0)


def _pool_tc(x_ref, batch_ref, wa_ref, ba_ref, wo_ref, bo_ref, out_ref):
    b = pl.program_id(0)
    x = x_ref[...]
    scores = _mmT(x, wa_ref[...]) + ba_ref[...]
    mask = batch_ref[...] == b
    s_i = jnp.where(mask, scores, -1e9)
    m = jnp.max(s_i, axis=0, keepdims=True)
    e = jnp.where(mask, jnp.exp(s_i - m), 0.0)
    denom = jnp.sum(e, axis=0, keepdims=True)
    w = e * (1.0 / jnp.maximum(denom, 1e-30))
    cvec = lax.dot_general(w, x, (((0,), (0,)), ((), ())),
                           preferred_element_type=jnp.float32)
    out_ref[0] = _mmT(cvec, wo_ref[...]) + bo_ref[...]


def _layer0_call(t0, t1, deg2, emb, wl, bl, wr, g, be):
    return pl.pallas_call(
        _layer0_tc,
        out_shape=(jax.ShapeDtypeStruct((_N, _D), jnp.float32),
                   jax.ShapeDtypeStruct((_N, 1), jnp.float32)),
    )(t0, t1, deg2, emb, wl, bl, wr, g, be)


def _layer1_call(x, parts, rvec, wl, bl, wr, g, be):
    return pl.pallas_call(
        _layer1_tc,
        out_shape=jax.ShapeDtypeStruct((_N, _D), jnp.float32),
    )(x, parts, rvec, wl, bl, wr, g, be)


def _pool_call(x, batch2, wa, ba, wo, bo):
    return pl.pallas_call(
        _pool_tc,
        grid=(_B,),
        in_specs=[
            pl.BlockSpec((_N, _D), lambda b: (0, 0)),
            pl.BlockSpec((_N, 1), lambda b: (0, 0)),
            pl.BlockSpec((_C, _D), lambda b: (0, 0)),
            pl.BlockSpec((1, _C), lambda b: (0, 0)),
            pl.BlockSpec((_D, _D), lambda b: (0, 0)),
            pl.BlockSpec((1, _D), lambda b: (0, 0)),
        ],
        out_specs=pl.BlockSpec((1, _C, _D), lambda b: (b, 0, 0)),
        out_shape=jax.ShapeDtypeStruct((_B, _C, _D), jnp.float32),
    )(x, batch2, wa, ba, wo, bo)


def kernel(deg_idx, edge_index, batch, emb, Wl0, bl0, Wr0, g0, be0,
           Wl1, bl1, Wr1, g1, be1, Wa, ba, Wo, bo):
    src = edge_index[0].astype(jnp.int32)
    dst = edge_index[1].astype(jnp.int32)
    npad = _EPAD - _E
    ppw = npad // _NW
    rpw = _E // _NW
    pad_src = (jnp.arange(npad, dtype=jnp.int32) * 37) % _N
    pad_dst = _TRASH + (jnp.arange(npad, dtype=jnp.int32) % (_NROWS - _N))
    src_f = jnp.concatenate([src.reshape(_NW, rpw),
                             pad_src.reshape(_NW, ppw)], axis=1).reshape(-1)
    dst_f = jnp.concatenate([dst.reshape(_NW, rpw),
                             pad_dst.reshape(_NW, ppw)], axis=1).reshape(-1)
    src_t = src_f.reshape(_NS, _TCH, _K)
    dst_t = dst_f.reshape(_NS, _TCH, _K)
    pk = jnp.stack([src_f.reshape(-1, _K), dst_f.reshape(-1, _K)],
                   axis=1)

    deg = deg_idx.astype(jnp.int32)
    deg2 = deg.reshape(_N, 1)
    batch2 = batch.astype(jnp.int32).reshape(_N, 1)
    bl0r = bl0.reshape(1, _D)
    g0r = g0.reshape(1, _D)
    be0r = be0.reshape(1, _D)
    bl1r = bl1.reshape(1, _D)
    g1r = g1.reshape(1, _D)
    be1r = be1.reshape(1, _D)
    bar = ba.reshape(1, _C)
    bor = bo.reshape(1, _D)

    t_flat = _hist_call()(deg, src_t, dst_t)
    t0 = t_flat[:_RH * _NV].reshape(_RH, _NV)
    t1 = t_flat[_TSZ:_TSZ + _RH * _NV].reshape(_RH, _NV)

    x1, rvec = _layer0_call(t0, t1, deg2, emb, Wl0, bl0r, Wr0, g0r, be0r)
    parts1 = _segsum_call()(x1, pk)
    x2 = _layer1_call(x1, parts1, rvec, Wl1, bl1r, Wr1, g1r, be1r)
    return _pool_call(x2, batch2, Wa, bar, Wo, bor)
